# baseline (speedup 1.0000x reference)
import jax
import jax.numpy as jnp
from jax import lax
from jax.experimental import pallas as pl
from jax.experimental.pallas import tpu as pltpu
from jax.experimental.pallas import tpu_sc as plsc

_B, _L, _D, _V = 4096, 200, 64, 1000000
_NC, _NS = 2, 16
_NW = _NC * _NS
_SPW = _B // _NW
_CHUNK = 104
_DH = _D // 2
_NREG = _DH // 16


def _sc_bow_body(x_hbm, table_hbm, out_hbm, idx_raw, rows_v, out_buf,
                 sem0, sem1):
    wid = lax.axis_index("s") * _NC + lax.axis_index("c")
    base = wid * _SPW
    pltpu.sync_copy(x_hbm.at[pl.ds(base, _SPW)], idx_raw)

    def descs(s, buf_ref, sem):
        a = pltpu.make_async_copy(
            table_hbm.at[idx_raw.at[s, pl.ds(0, _CHUNK)]],
            buf_ref.at[pl.ds(0, _CHUNK)], sem)
        b = pltpu.make_async_copy(
            table_hbm.at[idx_raw.at[s, pl.ds(_CHUNK, _L - _CHUNK)]],
            buf_ref.at[pl.ds(_CHUNK, _L - _CHUNK)], sem)
        return a, b

    def issue(s, buf_ref, sem):
        a, b = descs(s, buf_ref, sem)
        a.start()
        b.start()

    def drain(s, buf_ref, sem):
        a, b = descs(s, buf_ref, sem)
        a.wait()
        b.wait()

    def accum(s, buf_ref):
        def row_body(j, accs):
            return tuple(accs[d] + buf_ref[j, pl.ds(d * 16, 16)]
                         for d in range(_NREG))
        z = jnp.zeros((16,), jnp.float32)
        accs = lax.fori_loop(0, _L, row_body, (z,) * _NREG, unroll=8)
        for d in range(_NREG):
            out_buf[s, pl.ds(d * 16, 16)] = accs[d]

    buf0, buf1 = rows_v.at[0], rows_v.at[1]
    issue(0, buf0, sem0)

    def pair_body(g, carry):
        s0 = g * 2
        issue(s0 + 1, buf1, sem1)
        drain(s0, buf0, sem0)
        accum(s0, buf0)

        @pl.when(g < _SPW // 2 - 1)
        def _():
            issue(s0 + 2, buf0, sem0)

        drain(s0 + 1, buf1, sem1)
        accum(s0 + 1, buf1)
        return carry

    lax.fori_loop(0, _SPW // 2, pair_body, 0)
    pltpu.sync_copy(out_buf, out_hbm.at[pl.ds(base, _SPW)])


def _sc_bow(x, table_half):
    mesh = plsc.VectorSubcoreMesh(core_axis_name="c", subcore_axis_name="s")
    return pl.kernel(
        _sc_bow_body,
        mesh=mesh,
        compiler_params=pltpu.CompilerParams(use_tc_tiling_on_sc=False),
        out_type=jax.ShapeDtypeStruct((_B, _DH), jnp.float32),
        scratch_types=[
            pltpu.VMEM((_SPW, _L), jnp.int32),
            pltpu.VMEM((2, _L, _DH), jnp.float32),
            pltpu.VMEM((_SPW, _DH), jnp.float32),
            pltpu.SemaphoreType.DMA,
            pltpu.SemaphoreType.DMA,
        ],
    )(x, table_half)


def _tc_head_body(bowA_ref, bowB_ref, t_ref, W_hTA_ref, W_hTB_ref,
                  b_h_ref, gamma_ref, beta_ref,
                  W_o_ref, b_o_ref, loss_ref, logits_ref):
    inv = 1.0 / _L
    h_lin = (jnp.dot(bowA_ref[...] * inv, W_hTA_ref[...],
                     preferred_element_type=jnp.float32)
             + jnp.dot(bowB_ref[...] * inv, W_hTB_ref[...],
                       preferred_element_type=jnp.float32)
             + b_h_ref[...])
    mu = jnp.mean(h_lin, axis=0, keepdims=True)
    xc = h_lin - mu
    var = jnp.mean(xc * xc, axis=0, keepdims=True)
    h = xc * lax.rsqrt(var + 1e-5) * gamma_ref[...] + beta_ref[...]
    h = jnp.maximum(h, 0.0)
    logit = (jnp.sum(h * W_o_ref[...], axis=1, keepdims=True)
             + b_o_ref[...])
    t = t_ref[...]
    per = (jnp.maximum(logit, 0.0) - logit * t
           + jnp.log1p(jnp.exp(-jnp.abs(logit))))
    loss_ref[...] = jnp.mean(per, keepdims=True)
    logits_ref[...] = logit


def _tc_head(bowA, bowB, t, W_h, b_h, gamma, beta, W_o, b_o):
    W_hT = W_h.T
    return pl.pallas_call(
        _tc_head_body,
        out_shape=(jax.ShapeDtypeStruct((1, 1), jnp.float32),
                   jax.ShapeDtypeStruct((_B, 1), jnp.float32)),
    )(bowA, bowB, t.reshape(_B, 1), W_hT[:_DH], W_hT[_DH:],
      b_h.reshape(1, _D), gamma.reshape(1, _D), beta.reshape(1, _D),
      W_o, b_o.reshape(1, 1))


def kernel(x, t, table, W_h, b_h, gamma, beta, W_o, b_o):
    xi = x.astype(jnp.int32)
    bowA = _sc_bow(xi, table[:, :_DH])
    bowB = _sc_bow(xi, table[:, _DH:])
    loss2d, logits2d = _tc_head(bowA, bowB, t, W_h, b_h, gamma, beta,
                                W_o, b_o)
    return loss2d[0, 0], logits2d[:, 0]


# speedup vs baseline: 2.0483x; 2.0483x over previous
import jax
import jax.numpy as jnp
from jax import lax
from jax.experimental import pallas as pl
from jax.experimental.pallas import tpu as pltpu
from jax.experimental.pallas import tpu_sc as plsc

_B, _L, _D, _V = 4096, 200, 64, 1000000
_NC, _NS = 2, 16
_NW = _NC * _NS
_SPW = _B // _NW
_CHUNK = 104
_NREG = _D // 16


def _sc_bow_body(x_hbm, table_hbm, out_hbm, idx_raw, rows_v, out_buf,
                 sem0, sem1):
    wid = lax.axis_index("s") * _NC + lax.axis_index("c")
    base = wid * _SPW
    pltpu.sync_copy(x_hbm.at[pl.ds(base, _SPW)], idx_raw)

    def descs(s, buf_ref, sem):
        a = pltpu.make_async_copy(
            table_hbm.at[idx_raw.at[s, pl.ds(0, _CHUNK)]],
            buf_ref.at[pl.ds(0, _CHUNK)], sem)
        b = pltpu.make_async_copy(
            table_hbm.at[idx_raw.at[s, pl.ds(_CHUNK, _L - _CHUNK)]],
            buf_ref.at[pl.ds(_CHUNK, _L - _CHUNK)], sem)
        return a, b

    def issue(s, buf_ref, sem):
        a, b = descs(s, buf_ref, sem)
        a.start()
        b.start()

    def drain(s, buf_ref, sem):
        a, b = descs(s, buf_ref, sem)
        a.wait()
        b.wait()

    def accum(s, buf_ref):
        def row_body(j, accs):
            return tuple(accs[d] + buf_ref[j, pl.ds(d * 16, 16)]
                         for d in range(_NREG))
        z = jnp.zeros((16,), jnp.float32)
        accs = lax.fori_loop(0, _L, row_body, (z,) * _NREG, unroll=8)
        for d in range(_NREG):
            out_buf[s, pl.ds(d * 16, 16)] = accs[d]

    buf0, buf1 = rows_v.at[0], rows_v.at[1]
    issue(0, buf0, sem0)

    def pair_body(g, carry):
        s0 = g * 2
        issue(s0 + 1, buf1, sem1)
        drain(s0, buf0, sem0)
        accum(s0, buf0)

        @pl.when(g < _SPW // 2 - 1)
        def _():
            issue(s0 + 2, buf0, sem0)

        drain(s0 + 1, buf1, sem1)
        accum(s0 + 1, buf1)
        return carry

    lax.fori_loop(0, _SPW // 2, pair_body, 0)
    pltpu.sync_copy(out_buf, out_hbm.at[pl.ds(base, _SPW)])


def _sc_bow(x, table):
    mesh = plsc.VectorSubcoreMesh(core_axis_name="c", subcore_axis_name="s")
    return pl.kernel(
        _sc_bow_body,
        mesh=mesh,
        compiler_params=pltpu.CompilerParams(use_tc_tiling_on_sc=False),
        out_type=jax.ShapeDtypeStruct((_B, _D), jnp.float32),
        scratch_types=[
            pltpu.VMEM((_SPW, _L), jnp.int32),
            pltpu.VMEM((2, _L, _D), jnp.float32),
            pltpu.VMEM((_SPW, _D), jnp.float32),
            pltpu.SemaphoreType.DMA,
            pltpu.SemaphoreType.DMA,
        ],
    )(x, table)


def _tc_head_body(bow_ref, t_ref, W_hT_ref, b_h_ref, gamma_ref, beta_ref,
                  W_o_ref, b_o_ref, loss_ref, logits_ref):
    bow = bow_ref[...] * (1.0 / _L)
    h_lin = jnp.dot(bow, W_hT_ref[...],
                    preferred_element_type=jnp.float32) + b_h_ref[...]
    mu = jnp.mean(h_lin, axis=0, keepdims=True)
    xc = h_lin - mu
    var = jnp.mean(xc * xc, axis=0, keepdims=True)
    h = xc * lax.rsqrt(var + 1e-5) * gamma_ref[...] + beta_ref[...]
    h = jnp.maximum(h, 0.0)
    logit = (jnp.sum(h * W_o_ref[...], axis=1, keepdims=True)
             + b_o_ref[...])
    t = t_ref[...]
    per = (jnp.maximum(logit, 0.0) - logit * t
           + jnp.log1p(jnp.exp(-jnp.abs(logit))))
    loss_ref[...] = jnp.mean(per, keepdims=True)
    logits_ref[...] = logit


def _tc_head(bow_sum, t, W_h, b_h, gamma, beta, W_o, b_o):
    return pl.pallas_call(
        _tc_head_body,
        out_shape=(jax.ShapeDtypeStruct((1, 1), jnp.float32),
                   jax.ShapeDtypeStruct((_B, 1), jnp.float32)),
    )(bow_sum, t.reshape(_B, 1), W_h.T, b_h.reshape(1, _D),
      gamma.reshape(1, _D), beta.reshape(1, _D), W_o, b_o.reshape(1, 1))


def kernel(x, t, table, W_h, b_h, gamma, beta, W_o, b_o):
    bow_sum = _sc_bow(x.astype(jnp.int32), table)
    loss2d, logits2d = _tc_head(bow_sum, t, W_h, b_h, gamma, beta, W_o, b_o)
    return loss2d[0, 0], logits2d[:, 0]
